# baseline (speedup 1.0000x reference)
import functools

import numpy as np

import jax
import jax.numpy as jnp
from jax import lax
from jax.experimental import pallas as pl
from jax.experimental.pallas import tpu as pltpu
from jax.experimental.pallas import tpu_sc as plsc

B = 16384
F = 26
H = 16
V = 1000000
OUT = F + H
NC, NS, L = 2, 16, 16
NW = NC * NS
BPW = B // NW
C = 128
NCHUNK = BPW // C

W = 16384
_TGRID = (V + W - 1) // W
VP = _TGRID * W


M8 = W // 8

_P = np.zeros((8 * H, 128), np.float32)
for _m in range(8):
    for _h in range(H):
        _P[_m * H + _h, _m * H + _h] = 1.0


def _tp_body(in_ref, p_ref, o_ref):
    x = in_ref[...]
    p = p_ref[...]
    acc = None
    for m in range(8):
        xm = x[:, m * M8:(m + 1) * M8]
        d = jax.lax.dot_general(xm, p[m * H:(m + 1) * H, :],
                                (((0,), (0,)), ((), ())),
                                precision=jax.lax.Precision.HIGHEST,
                                preferred_element_type=jnp.float32)
        acc = d if acc is None else acc + d
    o_ref[...] = acc


_transpose_tc = pl.pallas_call(
    _tp_body,
    grid=(_TGRID,),
    in_specs=[pl.BlockSpec((H, W), lambda i: (0, i)),
              pl.BlockSpec((8 * H, 128), lambda i: (0, 0))],
    out_specs=pl.BlockSpec((W * H // 128, 128), lambda i: (i, 0)),
    out_shape=jax.ShapeDtypeStruct((VP * H // 128, 128), jnp.float32),
)

_mesh = plsc.VectorSubcoreMesh(core_axis_name="c", subcore_axis_name="s")


@functools.partial(
    pl.kernel,
    out_type=jax.ShapeDtypeStruct((OUT, B), jnp.float32),
    mesh=_mesh,
    compiler_params=pltpu.CompilerParams(needs_layout_passes=False,
                                         use_tc_tiling_on_sc=False),
    scratch_types=[
        pltpu.VMEM((F * C,), jnp.int32),
        pltpu.VMEM((F * C,), jnp.int32),
        pltpu.VMEM((F * C,), jnp.float32),
        pltpu.VMEM((F * C, H), jnp.float32),
        pltpu.VMEM((F * C,), jnp.float32),
        pltpu.VMEM((OUT, C), jnp.float32),
        pltpu.SemaphoreType.DMA,
        pltpu.SemaphoreType.DMA,
    ],
)
def _fm_sc(valT_hbm, idxT_hbm, tab_hbm, wtab_hbm, outT_hbm,
           idx_v, idxp_v, val_v, rows_v, w_v, out_v, sem_g, sem_io):
    wid = lax.axis_index("s") * NC + lax.axis_index("c")
    rowidx = F + lax.iota(jnp.int32, L)

    def chunk_body(chunk, _):
        b0 = wid * BPW + chunk * C
        stage_cps = [pltpu.async_copy(idxT_hbm.at[f, pl.ds(b0, C)],
                                      idx_v.at[pl.ds(f * C, C)], sem_io)
                     for f in range(F)]
        stage_cps += [pltpu.async_copy(valT_hbm.at[f, pl.ds(b0, C)],
                                       val_v.at[pl.ds(f * C, C)], sem_io)
                      for f in range(F)]
        for cp in stage_cps:
            cp.wait()

        def perm_body(i, _):
            v = idx_v[pl.ds(i * L, L)]
            idxp_v[pl.ds(i * L, L)] = ((v & ~(W - 1))
                                       | ((v & (M8 - 1)) << 3)
                                       | ((v >> 11) & 7))
            return 0

        lax.fori_loop(0, (F * C) // L, perm_body, 0)
        cp_rows = pltpu.async_copy(tab_hbm.at[idxp_v], rows_v, sem_g)
        cp_w = pltpu.async_copy(wtab_hbm.at[idx_v], w_v, sem_io)
        cp_rows.wait()
        cp_w.wait()

        def fm1_body(i, _):
            f = i // (C // L)
            c0 = (i % (C // L)) * L
            p0 = f * C + c0
            out_v[f, pl.ds(c0, L)] = w_v[pl.ds(p0, L)] * val_v[pl.ds(p0, L)]
            return 0

        lax.fori_loop(0, F * (C // L), fm1_body, 0)

        def fm2_body(q, _):
            cb = q * 4
            accs = [jnp.zeros((L,), jnp.float32) for _ in range(4)]
            acc2s = [jnp.zeros((L,), jnp.float32) for _ in range(4)]
            for f in range(F):
                for j in range(4):
                    p = f * C + cb + j
                    vv = plsc.load_gather(val_v, [jnp.full((L,), p, jnp.int32)])
                    ep = rows_v[p, :] * vv
                    accs[j] = accs[j] + ep
                    acc2s[j] = acc2s[j] + ep * ep
            for j in range(4):
                fm2 = 0.5 * (accs[j] * accs[j] - acc2s[j])
                plsc.store_scatter(out_v, [rowidx,
                                           jnp.full((L,), cb + j, jnp.int32)],
                                   fm2)
            return 0

        lax.fori_loop(0, C // 4, fm2_body, 0)
        pltpu.sync_copy(out_v, outT_hbm.at[:, pl.ds(b0, C)])
        return 0

    lax.fori_loop(0, NCHUNK, chunk_body, 0)


def kernel(feat_value, feat_index, embed_table, fm_1_weight_table):
    idxT = feat_index.astype(jnp.int32).T
    valT = feat_value.T
    tab_rm = _transpose_tc(embed_table.T, jnp.asarray(_P)).reshape(VP, H)
    outT = _fm_sc(valT, idxT, tab_rm, fm_1_weight_table)
    return outT.T


# speedup vs baseline: 1.5075x; 1.5075x over previous
import functools

import jax
import jax.numpy as jnp
from jax import lax
from jax.experimental import pallas as pl
from jax.experimental.pallas import tpu as pltpu
from jax.experimental.pallas import tpu_sc as plsc

B = 16384
F = 26
H = 16
V = 1000000
OUT = F + H
NC, NS, L = 2, 16, 16
NW = NC * NS
BPW = B // NW
C = 128
NCHUNK = BPW // C

W = 65536
_TGRID = (V + W - 1) // W
VP = _TGRID * W


M8 = W // 8
_LOG_M8 = M8.bit_length() - 1


def _tp_body(in_ref, o_ref):
    t = in_ref[...].T
    o_ref[...] = jnp.concatenate(
        [t[m * M8:(m + 1) * M8, :] for m in range(8)], axis=1)


_transpose_tc = pl.pallas_call(
    _tp_body,
    grid=(_TGRID,),
    in_specs=[pl.BlockSpec((H, W), lambda i: (0, i))],
    out_specs=pl.BlockSpec((W * H // 128, 128), lambda i: (i, 0)),
    out_shape=jax.ShapeDtypeStruct((VP * H // 128, 128), jnp.float32),
)

_mesh = plsc.VectorSubcoreMesh(core_axis_name="c", subcore_axis_name="s")


@functools.partial(
    pl.kernel,
    out_type=jax.ShapeDtypeStruct((OUT, B), jnp.float32),
    mesh=_mesh,
    compiler_params=pltpu.CompilerParams(needs_layout_passes=False,
                                         use_tc_tiling_on_sc=False),
    scratch_types=[
        pltpu.VMEM((F * C,), jnp.int32),
        pltpu.VMEM((F * C,), jnp.int32),
        pltpu.VMEM((F * C,), jnp.float32),
        pltpu.VMEM((F * C, H), jnp.float32),
        pltpu.VMEM((F * C,), jnp.float32),
        pltpu.VMEM((OUT, C), jnp.float32),
        pltpu.SemaphoreType.DMA,
        pltpu.SemaphoreType.DMA,
    ],
)
def _fm_sc(valT_hbm, idxT_hbm, tab_hbm, wtab_hbm, outT_hbm,
           idx_v, idxp_v, val_v, rows_v, w_v, out_v, sem_g, sem_io):
    wid = lax.axis_index("s") * NC + lax.axis_index("c")
    rowidx = F + lax.iota(jnp.int32, L)

    def chunk_body(chunk, _):
        b0 = wid * BPW + chunk * C
        stage_cps = [pltpu.async_copy(idxT_hbm.at[f, pl.ds(b0, C)],
                                      idx_v.at[pl.ds(f * C, C)], sem_io)
                     for f in range(F)]
        stage_cps += [pltpu.async_copy(valT_hbm.at[f, pl.ds(b0, C)],
                                       val_v.at[pl.ds(f * C, C)], sem_io)
                      for f in range(F)]
        for cp in stage_cps:
            cp.wait()

        def perm_body(i, _):
            v = idx_v[pl.ds(i * L, L)]
            idxp_v[pl.ds(i * L, L)] = ((v & ~(W - 1))
                                       | ((v & (M8 - 1)) << 3)
                                       | ((v >> _LOG_M8) & 7))
            return 0

        lax.fori_loop(0, (F * C) // L, perm_body, 0)
        cp_rows = pltpu.async_copy(tab_hbm.at[idxp_v], rows_v, sem_g)
        cp_w = pltpu.async_copy(wtab_hbm.at[idx_v], w_v, sem_io)
        cp_rows.wait()
        cp_w.wait()

        def fm1_body(i, _):
            f = i // (C // L)
            c0 = (i % (C // L)) * L
            p0 = f * C + c0
            out_v[f, pl.ds(c0, L)] = w_v[pl.ds(p0, L)] * val_v[pl.ds(p0, L)]
            return 0

        lax.fori_loop(0, F * (C // L), fm1_body, 0)

        def fm2_body(q, _):
            cb = q * 4
            accs = [jnp.zeros((L,), jnp.float32) for _ in range(4)]
            acc2s = [jnp.zeros((L,), jnp.float32) for _ in range(4)]
            for f in range(F):
                for j in range(4):
                    p = f * C + cb + j
                    vv = plsc.load_gather(val_v, [jnp.full((L,), p, jnp.int32)])
                    ep = rows_v[p, :] * vv
                    accs[j] = accs[j] + ep
                    acc2s[j] = acc2s[j] + ep * ep
            for j in range(4):
                fm2 = 0.5 * (accs[j] * accs[j] - acc2s[j])
                plsc.store_scatter(out_v, [rowidx,
                                           jnp.full((L,), cb + j, jnp.int32)],
                                   fm2)
            return 0

        lax.fori_loop(0, C // 4, fm2_body, 0)
        pltpu.sync_copy(out_v, outT_hbm.at[:, pl.ds(b0, C)])
        return 0

    lax.fori_loop(0, NCHUNK, chunk_body, 0)


def kernel(feat_value, feat_index, embed_table, fm_1_weight_table):
    idxT = feat_index.astype(jnp.int32).T
    valT = feat_value.T
    tab_rm = _transpose_tc(embed_table.T).reshape(VP, H)
    outT = _fm_sc(valT, idxT, tab_rm, fm_1_weight_table)
    return outT.T


# speedup vs baseline: 1.5353x; 1.0184x over previous
import functools

import jax
import jax.numpy as jnp
from jax import lax
from jax.experimental import pallas as pl
from jax.experimental.pallas import tpu as pltpu
from jax.experimental.pallas import tpu_sc as plsc

B = 16384
F = 26
H = 16
V = 1000000
OUT = F + H
NC, NS, L = 2, 16, 16
NW = NC * NS
BPW = B // NW
C = 128
NCHUNK = BPW // C

W = 16384
_TGRID = (V + W - 1) // W
VP = _TGRID * W


M8 = W // 8
_LOG_M8 = M8.bit_length() - 1


def _tp_body(in_ref, o_ref):
    t = in_ref[...].T
    o_ref[...] = jnp.concatenate(
        [t[m * M8:(m + 1) * M8, :] for m in range(8)], axis=1)


_transpose_tc = pl.pallas_call(
    _tp_body,
    grid=(_TGRID,),
    in_specs=[pl.BlockSpec((H, W), lambda i: (0, i))],
    out_specs=pl.BlockSpec((W * H // 128, 128), lambda i: (i, 0)),
    out_shape=jax.ShapeDtypeStruct((VP * H // 128, 128), jnp.float32),
)

_mesh = plsc.VectorSubcoreMesh(core_axis_name="c", subcore_axis_name="s")


@functools.partial(
    pl.kernel,
    out_type=jax.ShapeDtypeStruct((OUT, B), jnp.float32),
    mesh=_mesh,
    compiler_params=pltpu.CompilerParams(needs_layout_passes=False,
                                         use_tc_tiling_on_sc=False),
    scratch_types=[
        pltpu.VMEM((F * C,), jnp.int32),
        pltpu.VMEM((F * C,), jnp.int32),
        pltpu.VMEM((F * C,), jnp.float32),
        pltpu.VMEM((F * C, H), jnp.float32),
        pltpu.VMEM((F * C,), jnp.float32),
        pltpu.VMEM((OUT, C), jnp.float32),
        pltpu.SemaphoreType.DMA,
        pltpu.SemaphoreType.DMA,
    ],
)
def _fm_sc(valT_hbm, idxT_hbm, tab_hbm, wtab_hbm, outT_hbm,
           idx_v, idxp_v, val_v, rows_v, w_v, out_v, sem_g, sem_io):
    wid = lax.axis_index("s") * NC + lax.axis_index("c")
    rowidx = F + lax.iota(jnp.int32, L)

    def chunk_body(chunk, _):
        b0 = wid * BPW + chunk * C
        stage_cps = [pltpu.async_copy(idxT_hbm.at[f, pl.ds(b0, C)],
                                      idx_v.at[pl.ds(f * C, C)], sem_io)
                     for f in range(F)]
        stage_cps += [pltpu.async_copy(valT_hbm.at[f, pl.ds(b0, C)],
                                       val_v.at[pl.ds(f * C, C)], sem_io)
                      for f in range(F)]
        for cp in stage_cps:
            cp.wait()

        def perm_body(i, _):
            v = idx_v[pl.ds(i * L, L)]
            idxp_v[pl.ds(i * L, L)] = ((v & ~(W - 1))
                                       | ((v & (M8 - 1)) << 3)
                                       | ((v >> _LOG_M8) & 7))
            return 0

        lax.fori_loop(0, (F * C) // L, perm_body, 0)
        cp_rows = pltpu.async_copy(tab_hbm.at[idxp_v], rows_v, sem_g)
        cp_w = pltpu.async_copy(wtab_hbm.at[idx_v], w_v, sem_io)
        cp_rows.wait()
        cp_w.wait()

        def fm1_body(i, _):
            f = i // (C // L)
            c0 = (i % (C // L)) * L
            p0 = f * C + c0
            out_v[f, pl.ds(c0, L)] = w_v[pl.ds(p0, L)] * val_v[pl.ds(p0, L)]
            return 0

        lax.fori_loop(0, F * (C // L), fm1_body, 0)

        def fm2_body(q, _):
            cb = q * 4
            accs = [jnp.zeros((L,), jnp.float32) for _ in range(4)]
            acc2s = [jnp.zeros((L,), jnp.float32) for _ in range(4)]
            for f in range(F):
                for j in range(4):
                    p = f * C + cb + j
                    vv = plsc.load_gather(val_v, [jnp.full((L,), p, jnp.int32)])
                    ep = rows_v[p, :] * vv
                    accs[j] = accs[j] + ep
                    acc2s[j] = acc2s[j] + ep * ep
            for j in range(4):
                fm2 = 0.5 * (accs[j] * accs[j] - acc2s[j])
                plsc.store_scatter(out_v, [rowidx,
                                           jnp.full((L,), cb + j, jnp.int32)],
                                   fm2)
            return 0

        lax.fori_loop(0, C // 4, fm2_body, 0)
        pltpu.sync_copy(out_v, outT_hbm.at[:, pl.ds(b0, C)])
        return 0

    lax.fori_loop(0, NCHUNK, chunk_body, 0)


def kernel(feat_value, feat_index, embed_table, fm_1_weight_table):
    idxT = feat_index.astype(jnp.int32).T
    valT = feat_value.T
    tab_rm = _transpose_tc(embed_table.T).reshape(VP, H)
    outT = _fm_sc(valT, idxT, tab_rm, fm_1_weight_table)
    return outT.T
